# baseline (speedup 1.0000x reference)
import jax
import jax.numpy as jnp
from jax.experimental import pallas as pl
from jax.experimental.pallas import tpu as pltpu

V = 100000
VP = 100096
R = 8


def _bitcast_f32(x):
    return jax.lax.bitcast_convert_type(x, jnp.float32)


def _body(l_ref, t_ref, p_ref, u_ref, out_ref, s_ref):
    t = jnp.maximum(t_ref[...][:, :1], 1e-8)
    s_ref[...] = l_ref[...] / t
    s = s_ref[...]
    m = jnp.max(s, axis=-1, keepdims=True)
    w = jnp.exp(s - m)
    zw = jnp.sum(w, axis=-1, keepdims=True)

    def mass_gt(v):
        return jnp.sum(w * (w > v).astype(jnp.float32),
                       axis=-1, keepdims=True)

    def snap(v):
        return jnp.max(jnp.where(w <= v, w, 0.0), axis=-1, keepdims=True)

    def search(target, strict):
        def step(_, carry):
            lo, hi = carry
            mid = jax.lax.shift_right_logical(lo + hi, 1)
            msum = mass_gt(_bitcast_f32(mid))
            cond = (msum < target) if strict else (msum <= target)
            return jnp.where(cond, lo, mid + 1), jnp.where(cond, mid, hi)

        lo0 = jnp.zeros((R, 1), jnp.int32)
        hi0 = jnp.full((R, 1), 0x3FC00000, jnp.int32)
        lo, _ = jax.lax.fori_loop(0, 30, step, (lo0, hi0))
        return _bitcast_f32(lo)

    target1 = p_ref[...][:, :1] * zw
    v1 = snap(search(target1, strict=False))
    ma1 = mass_gt(v1)
    ce1 = jnp.sum(jnp.where(w == v1, 1.0, 0.0), axis=-1, keepdims=True)
    r1 = jnp.floor((target1 - ma1) / jnp.maximum(v1, 1e-45)) + 1.0
    r1 = jnp.clip(r1, 1.0, ce1)
    r1 = jnp.where(v1 > 0, r1, ce1)
    s_kept = ma1 + r1 * v1

    target2 = u_ref[...][:, :1] * s_kept
    v2 = snap(search(target2, strict=True))
    ma2 = mass_gt(v2)
    ce2 = jnp.sum(jnp.where(w == v2, 1.0, 0.0), axis=-1, keepdims=True)
    r2 = jnp.ceil((target2 - ma2) / jnp.maximum(v2, 1e-45))
    r2 = jnp.where(ma2 + r2 * v2 < target2, r2 + 1.0, r2)
    r2 = jnp.clip(r2, 1.0, ce2)
    swap = v2 < v1
    v2 = jnp.where(swap, v1, v2)
    r2 = jnp.where(swap, r1, jnp.where(v2 == v1, jnp.minimum(r2, r1), r2))

    eq = w == v2
    idx = jax.lax.broadcasted_iota(jnp.int32, (R, VP), 1)

    def istep(_, carry):
        lo, hi = carry
        mid = jax.lax.shift_right_logical(lo + hi, 1)
        cnt = jnp.sum(jnp.where(eq & (idx <= mid), 1.0, 0.0),
                      axis=-1, keepdims=True)
        cond = cnt >= r2
        return jnp.where(cond, lo, mid + 1), jnp.where(cond, mid, hi)

    lo0 = jnp.zeros((R, 1), jnp.int32)
    hi0 = jnp.full((R, 1), V - 1, jnp.int32)
    lo, _ = jax.lax.fori_loop(0, 17, istep, (lo0, hi0))
    out_ref[...] = jnp.broadcast_to(lo, (R, 128))


def kernel(logits, temperature, top_p):
    n = logits.shape[0]
    lp = jnp.pad(logits, ((0, 0), (0, VP - V)), constant_values=-1e30)
    u = jax.random.uniform(jax.random.key(42), (n, 1), dtype=jnp.float32)
    tb = jnp.broadcast_to(temperature, (n, 128))
    pb = jnp.broadcast_to(top_p, (n, 128))
    ub = jnp.broadcast_to(u, (n, 128))
    out = pl.pallas_call(
        _body,
        grid=(n // R,),
        in_specs=[
            pl.BlockSpec((R, VP), lambda i: (i, 0)),
            pl.BlockSpec((R, 128), lambda i: (i, 0)),
            pl.BlockSpec((R, 128), lambda i: (i, 0)),
            pl.BlockSpec((R, 128), lambda i: (i, 0)),
        ],
        out_specs=pl.BlockSpec((R, 128), lambda i: (i, 0)),
        out_shape=jax.ShapeDtypeStruct((n, 128), jnp.int32),
        scratch_shapes=[pltpu.VMEM((R, VP), jnp.float32)],
        compiler_params=pltpu.CompilerParams(
            dimension_semantics=("parallel",)),
    )(lp, tb, pb, ub)
    return out[:, :1]


# speedup vs baseline: 1.4530x; 1.4530x over previous
import jax
import jax.numpy as jnp
from jax.experimental import pallas as pl
from jax.experimental.pallas import tpu as pltpu

V = 100000
VP = 100096
R = 8


def _bitcast_f32(x):
    return jax.lax.bitcast_convert_type(x, jnp.float32)


def _body(l_ref, t_ref, p_ref, u_ref, out_ref, s_ref):
    t = jnp.maximum(t_ref[...][:, :1], 1e-8)
    s_ref[...] = l_ref[...] / t
    s = s_ref[...]

    _B = (25088, 50176, 75264)

    def rsum(x):
        p0 = jnp.sum(x[:, :_B[0]], axis=-1, keepdims=True)
        p1 = jnp.sum(x[:, _B[0]:_B[1]], axis=-1, keepdims=True)
        p2 = jnp.sum(x[:, _B[1]:_B[2]], axis=-1, keepdims=True)
        p3 = jnp.sum(x[:, _B[2]:], axis=-1, keepdims=True)
        return (p0 + p1) + (p2 + p3)

    def rmax(x):
        p0 = jnp.max(x[:, :_B[0]], axis=-1, keepdims=True)
        p1 = jnp.max(x[:, _B[0]:_B[1]], axis=-1, keepdims=True)
        p2 = jnp.max(x[:, _B[1]:_B[2]], axis=-1, keepdims=True)
        p3 = jnp.max(x[:, _B[2]:], axis=-1, keepdims=True)
        return jnp.maximum(jnp.maximum(p0, p1), jnp.maximum(p2, p3))

    m = rmax(s)
    w = jnp.exp(s - m)
    zw = rsum(w)

    def mass_gt(v):
        return rsum(w * (w > v).astype(jnp.float32))

    def snap(v):
        return rmax(jnp.where(w <= v, w, 0.0))

    def search(target, strict):
        def step(_, carry):
            lo, hi = carry
            mid = jax.lax.shift_right_logical(lo + hi, 1)
            msum = mass_gt(_bitcast_f32(mid))
            cond = (msum < target) if strict else (msum <= target)
            return jnp.where(cond, lo, mid + 1), jnp.where(cond, mid, hi)

        lo0 = jnp.zeros((R, 1), jnp.int32)
        hi0 = jnp.full((R, 1), 0x3FC00000, jnp.int32)
        lo, _ = jax.lax.fori_loop(0, 30, step, (lo0, hi0))
        return _bitcast_f32(lo)

    target1 = p_ref[...][:, :1] * zw
    v1 = snap(search(target1, strict=False))
    ma1 = mass_gt(v1)
    ce1 = rsum(jnp.where(w == v1, 1.0, 0.0))
    r1 = jnp.floor((target1 - ma1) / jnp.maximum(v1, 1e-45)) + 1.0
    r1 = jnp.clip(r1, 1.0, ce1)
    r1 = jnp.where(v1 > 0, r1, ce1)
    s_kept = ma1 + r1 * v1

    target2 = u_ref[...][:, :1] * s_kept
    v2 = snap(search(target2, strict=True))
    ma2 = mass_gt(v2)
    ce2 = rsum(jnp.where(w == v2, 1.0, 0.0))
    r2 = jnp.ceil((target2 - ma2) / jnp.maximum(v2, 1e-45))
    r2 = jnp.where(ma2 + r2 * v2 < target2, r2 + 1.0, r2)
    r2 = jnp.clip(r2, 1.0, ce2)
    swap = v2 < v1
    v2 = jnp.where(swap, v1, v2)
    r2 = jnp.where(swap, r1, jnp.where(v2 == v1, jnp.minimum(r2, r1), r2))

    eq = w == v2
    idx = jax.lax.broadcasted_iota(jnp.int32, (R, VP), 1)

    def istep(_, carry):
        lo, hi = carry
        mid = jax.lax.shift_right_logical(lo + hi, 1)
        cnt = rsum(jnp.where(eq & (idx <= mid), 1.0, 0.0))
        cond = cnt >= r2
        return jnp.where(cond, lo, mid + 1), jnp.where(cond, mid, hi)

    lo0 = jnp.zeros((R, 1), jnp.int32)
    hi0 = jnp.full((R, 1), V - 1, jnp.int32)
    lo, _ = jax.lax.fori_loop(0, 17, istep, (lo0, hi0))
    out_ref[...] = jnp.broadcast_to(lo, (R, 128))


def kernel(logits, temperature, top_p):
    n = logits.shape[0]
    lp = jnp.pad(logits, ((0, 0), (0, VP - V)), constant_values=-1e30)
    u = jax.random.uniform(jax.random.key(42), (n, 1), dtype=jnp.float32)
    tb = jnp.broadcast_to(temperature, (n, 128))
    pb = jnp.broadcast_to(top_p, (n, 128))
    ub = jnp.broadcast_to(u, (n, 128))
    out = pl.pallas_call(
        _body,
        grid=(n // R,),
        in_specs=[
            pl.BlockSpec((R, VP), lambda i: (i, 0)),
            pl.BlockSpec((R, 128), lambda i: (i, 0)),
            pl.BlockSpec((R, 128), lambda i: (i, 0)),
            pl.BlockSpec((R, 128), lambda i: (i, 0)),
        ],
        out_specs=pl.BlockSpec((R, 128), lambda i: (i, 0)),
        out_shape=jax.ShapeDtypeStruct((n, 128), jnp.int32),
        scratch_shapes=[pltpu.VMEM((R, VP), jnp.float32)],
        compiler_params=pltpu.CompilerParams(
            dimension_semantics=("parallel",)),
    )(lp, tb, pb, ub)
    return out[:, :1]


# speedup vs baseline: 1.5690x; 1.0798x over previous
import jax
import jax.numpy as jnp
from jax.experimental import pallas as pl
from jax.experimental.pallas import tpu as pltpu

V = 100000
VP = 100096
R = 8


def _bitcast_f32(x):
    return jax.lax.bitcast_convert_type(x, jnp.float32)


def _body(l_ref, t_ref, p_ref, u_ref, out_ref, s_ref):
    t = jnp.maximum(t_ref[...][:, :1], 1e-8)
    s_ref[...] = l_ref[...] / t
    s = s_ref[...]

    _B = tuple(12544 * k for k in range(1, 8)) + (VP,)

    def _tree(parts, op):
        while len(parts) > 1:
            parts = [op(parts[i], parts[i + 1]) if i + 1 < len(parts)
                     else parts[i] for i in range(0, len(parts), 2)]
        return parts[0]

    def rsum(x):
        lo = 0
        parts = []
        for hi in _B:
            parts.append(jnp.sum(x[:, lo:hi], axis=-1, keepdims=True))
            lo = hi
        return _tree(parts, jnp.add)

    def rmax(x):
        lo = 0
        parts = []
        for hi in _B:
            parts.append(jnp.max(x[:, lo:hi], axis=-1, keepdims=True))
            lo = hi
        return _tree(parts, jnp.maximum)

    m = rmax(s)
    w = jnp.exp(s - m)
    zw = rsum(w)

    def mass_gt(v):
        return rsum(w * (w > v).astype(jnp.float32))

    def snap(v):
        return rmax(jnp.where(w <= v, w, 0.0))

    def search(target, strict):
        def step(_, carry):
            lo, hi = carry
            mid = jax.lax.shift_right_logical(lo + hi, 1)
            msum = mass_gt(_bitcast_f32(mid))
            cond = (msum < target) if strict else (msum <= target)
            return jnp.where(cond, lo, mid + 1), jnp.where(cond, mid, hi)

        lo0 = jnp.zeros((R, 1), jnp.int32)
        hi0 = jnp.full((R, 1), 0x3FC00000, jnp.int32)
        lo, _ = jax.lax.fori_loop(0, 30, step, (lo0, hi0))
        return _bitcast_f32(lo)

    target1 = p_ref[...][:, :1] * zw
    v1 = snap(search(target1, strict=False))
    ma1 = mass_gt(v1)
    ce1 = rsum(jnp.where(w == v1, 1.0, 0.0))
    r1 = jnp.floor((target1 - ma1) / jnp.maximum(v1, 1e-45)) + 1.0
    r1 = jnp.clip(r1, 1.0, ce1)
    r1 = jnp.where(v1 > 0, r1, ce1)
    s_kept = ma1 + r1 * v1

    target2 = u_ref[...][:, :1] * s_kept
    v2 = snap(search(target2, strict=True))
    ma2 = mass_gt(v2)
    ce2 = rsum(jnp.where(w == v2, 1.0, 0.0))
    r2 = jnp.ceil((target2 - ma2) / jnp.maximum(v2, 1e-45))
    r2 = jnp.where(ma2 + r2 * v2 < target2, r2 + 1.0, r2)
    r2 = jnp.clip(r2, 1.0, ce2)
    swap = v2 < v1
    v2 = jnp.where(swap, v1, v2)
    r2 = jnp.where(swap, r1, jnp.where(v2 == v1, jnp.minimum(r2, r1), r2))

    eq = w == v2
    idx = jax.lax.broadcasted_iota(jnp.int32, (R, VP), 1)

    def istep(_, carry):
        lo, hi = carry
        mid = jax.lax.shift_right_logical(lo + hi, 1)
        cnt = rsum(jnp.where(eq & (idx <= mid), 1.0, 0.0))
        cond = cnt >= r2
        return jnp.where(cond, lo, mid + 1), jnp.where(cond, mid, hi)

    lo0 = jnp.zeros((R, 1), jnp.int32)
    hi0 = jnp.full((R, 1), V - 1, jnp.int32)
    lo, _ = jax.lax.fori_loop(0, 17, istep, (lo0, hi0))
    out_ref[...] = jnp.broadcast_to(lo, (R, 128))


def kernel(logits, temperature, top_p):
    n = logits.shape[0]
    lp = jnp.pad(logits, ((0, 0), (0, VP - V)), constant_values=-1e30)
    u = jax.random.uniform(jax.random.key(42), (n, 1), dtype=jnp.float32)
    tb = jnp.broadcast_to(temperature, (n, 128))
    pb = jnp.broadcast_to(top_p, (n, 128))
    ub = jnp.broadcast_to(u, (n, 128))
    out = pl.pallas_call(
        _body,
        grid=(n // R,),
        in_specs=[
            pl.BlockSpec((R, VP), lambda i: (i, 0)),
            pl.BlockSpec((R, 128), lambda i: (i, 0)),
            pl.BlockSpec((R, 128), lambda i: (i, 0)),
            pl.BlockSpec((R, 128), lambda i: (i, 0)),
        ],
        out_specs=pl.BlockSpec((R, 128), lambda i: (i, 0)),
        out_shape=jax.ShapeDtypeStruct((n, 128), jnp.int32),
        scratch_shapes=[pltpu.VMEM((R, VP), jnp.float32)],
        compiler_params=pltpu.CompilerParams(
            dimension_semantics=("parallel",)),
    )(lp, tb, pb, ub)
    return out[:, :1]


# speedup vs baseline: 1.6274x; 1.0372x over previous
import jax
import jax.numpy as jnp
from jax.experimental import pallas as pl
from jax.experimental.pallas import tpu as pltpu

V = 100000
VP = 100096
R = 8


def _bitcast_f32(x):
    return jax.lax.bitcast_convert_type(x, jnp.float32)


def _body(l_ref, t_ref, p_ref, u_ref, out_ref, s_ref):
    t = jnp.maximum(t_ref[...][:, :1], 1e-8)
    s_ref[...] = l_ref[...] / t
    s = s_ref[...]

    _B = tuple(6272 * k for k in range(1, 16)) + (VP,)

    def _tree(parts, op):
        while len(parts) > 1:
            parts = [op(parts[i], parts[i + 1]) if i + 1 < len(parts)
                     else parts[i] for i in range(0, len(parts), 2)]
        return parts[0]

    def rsum(x):
        lo = 0
        parts = []
        for hi in _B:
            parts.append(jnp.sum(x[:, lo:hi], axis=-1, keepdims=True))
            lo = hi
        return _tree(parts, jnp.add)

    def rmax(x):
        lo = 0
        parts = []
        for hi in _B:
            parts.append(jnp.max(x[:, lo:hi], axis=-1, keepdims=True))
            lo = hi
        return _tree(parts, jnp.maximum)

    m = rmax(s)
    w = jnp.exp(s - m)
    zw = rsum(w)

    def mass_gt(v):
        return rsum(w * (w > v).astype(jnp.float32))

    def snap(v):
        return rmax(jnp.where(w <= v, w, 0.0))

    def search(target, strict):
        def step(_, carry):
            lo, hi = carry
            mid = jax.lax.shift_right_logical(lo + hi, 1)
            msum = mass_gt(_bitcast_f32(mid))
            cond = (msum < target) if strict else (msum <= target)
            return jnp.where(cond, lo, mid + 1), jnp.where(cond, mid, hi)

        lo0 = jnp.zeros((R, 1), jnp.int32)
        hi0 = jnp.full((R, 1), 0x3FC00000, jnp.int32)
        lo, _ = jax.lax.fori_loop(0, 30, step, (lo0, hi0))
        return _bitcast_f32(lo)

    target1 = p_ref[...][:, :1] * zw
    v1 = snap(search(target1, strict=False))
    ma1 = mass_gt(v1)
    ce1 = rsum(jnp.where(w == v1, 1.0, 0.0))
    r1 = jnp.floor((target1 - ma1) / jnp.maximum(v1, 1e-45)) + 1.0
    r1 = jnp.clip(r1, 1.0, ce1)
    r1 = jnp.where(v1 > 0, r1, ce1)
    s_kept = ma1 + r1 * v1

    target2 = u_ref[...][:, :1] * s_kept
    v2 = snap(search(target2, strict=True))
    ma2 = mass_gt(v2)
    ce2 = rsum(jnp.where(w == v2, 1.0, 0.0))
    r2 = jnp.ceil((target2 - ma2) / jnp.maximum(v2, 1e-45))
    r2 = jnp.where(ma2 + r2 * v2 < target2, r2 + 1.0, r2)
    r2 = jnp.clip(r2, 1.0, ce2)
    swap = v2 < v1
    v2 = jnp.where(swap, v1, v2)
    r2 = jnp.where(swap, r1, jnp.where(v2 == v1, jnp.minimum(r2, r1), r2))

    eq = w == v2
    idx = jax.lax.broadcasted_iota(jnp.int32, (R, VP), 1)

    def istep(_, carry):
        lo, hi = carry
        mid = jax.lax.shift_right_logical(lo + hi, 1)
        cnt = rsum(jnp.where(eq & (idx <= mid), 1.0, 0.0))
        cond = cnt >= r2
        return jnp.where(cond, lo, mid + 1), jnp.where(cond, mid, hi)

    lo0 = jnp.zeros((R, 1), jnp.int32)
    hi0 = jnp.full((R, 1), V - 1, jnp.int32)
    lo, _ = jax.lax.fori_loop(0, 17, istep, (lo0, hi0))
    out_ref[...] = jnp.broadcast_to(lo, (R, 128))


def kernel(logits, temperature, top_p):
    n = logits.shape[0]
    lp = jnp.pad(logits, ((0, 0), (0, VP - V)), constant_values=-1e30)
    u = jax.random.uniform(jax.random.key(42), (n, 1), dtype=jnp.float32)
    tb = jnp.broadcast_to(temperature, (n, 128))
    pb = jnp.broadcast_to(top_p, (n, 128))
    ub = jnp.broadcast_to(u, (n, 128))
    out = pl.pallas_call(
        _body,
        grid=(n // R,),
        in_specs=[
            pl.BlockSpec((R, VP), lambda i: (i, 0)),
            pl.BlockSpec((R, 128), lambda i: (i, 0)),
            pl.BlockSpec((R, 128), lambda i: (i, 0)),
            pl.BlockSpec((R, 128), lambda i: (i, 0)),
        ],
        out_specs=pl.BlockSpec((R, 128), lambda i: (i, 0)),
        out_shape=jax.ShapeDtypeStruct((n, 128), jnp.int32),
        scratch_shapes=[pltpu.VMEM((R, VP), jnp.float32)],
        compiler_params=pltpu.CompilerParams(
            dimension_semantics=("parallel",)),
    )(lp, tb, pb, ub)
    return out[:, :1]
